# baseline (speedup 1.0000x reference)
import functools

import jax
import jax.numpy as jnp
from jax import lax
from jax.experimental import pallas as pl
from jax.experimental.pallas import tpu as pltpu
from jax.experimental.pallas import tpu_sc as plsc

LANES = 16
NC, NS = 2, 16
NW = NC * NS
NBB = 8


def _tpe_body(x_hbm, tokT_hbm, posT_hbm, outT_hbm, row_v, posrow_v, xblk,
              oblk, x_sems, o_sems, ld_sem, *, batch, seq, dmodel):
    wid = lax.axis_index("s") * NC + lax.axis_index("c")
    nblk = batch // NBB
    npass = dmodel // NW

    def x_copy(blk, par):
        return pltpu.make_async_copy(x_hbm.at[pl.ds(blk * NBB, NBB)],
                                     xblk[par], x_sems[par])

    for p in range(npass):
        d = wid + p * NW

        pltpu.async_copy(tokT_hbm.at[d], row_v, ld_sem)
        pltpu.make_async_copy(tokT_hbm.at[d], row_v, ld_sem).wait()
        pltpu.async_copy(posT_hbm.at[d], posrow_v, ld_sem)
        pltpu.make_async_copy(posT_hbm.at[d], posrow_v, ld_sem).wait()

        def o_copy(blk, par, d=d):
            return pltpu.make_async_copy(
                oblk[par], outT_hbm.at[pl.ds(blk * NBB, NBB), d],
                o_sems[par])

        x_copy(0, 0).start()

        @pl.loop(0, nblk, step=2)
        def _pair(blk0, o_copy=o_copy):
            for par in range(2):
                blk = blk0 + par

                @pl.when(blk + 1 < nblk)
                def _prefetch(blk=blk, par=par):
                    x_copy(blk + 1, 1 - par).start()

                x_copy(blk, par).wait()

                @pl.when(blk - 2 >= 0)
                def _drain(blk=blk, par=par, o_copy=o_copy):
                    o_copy(blk - 2, par).wait()

                @pl.loop(0, seq // LANES)
                def _sc(sc, par=par):
                    pv = posrow_v[pl.ds(sc * LANES, LANES)]
                    for bb in range(NBB):
                        idx = xblk[par][bb, pl.ds(sc * LANES, LANES)]
                        g = plsc.load_gather(row_v, [idx])
                        oblk[par][bb, pl.ds(sc * LANES, LANES)] = g + pv

                o_copy(blk, par).start()

        for par in range(2):
            o_copy(nblk - 2 + par, par).wait()


def kernel(x, token_table, pos_table):
    batch, seq = x.shape
    vocab, dmodel = token_table.shape

    idx = x.astype(jnp.int32)
    tokT = token_table.T
    posT = pos_table.T
    mesh = plsc.VectorSubcoreMesh(core_axis_name="c", subcore_axis_name="s")

    run = pl.kernel(
        functools.partial(_tpe_body, batch=batch, seq=seq, dmodel=dmodel),
        out_type=jax.ShapeDtypeStruct((batch, dmodel, seq), jnp.float32),
        mesh=mesh,
        scratch_types=[
            pltpu.VMEM((vocab,), jnp.float32),
            pltpu.VMEM((seq,), jnp.float32),
            [pltpu.VMEM((NBB, seq), jnp.int32) for _ in range(2)],
            [pltpu.VMEM((NBB, seq), jnp.float32) for _ in range(2)],
            [pltpu.SemaphoreType.DMA for _ in range(2)],
            [pltpu.SemaphoreType.DMA for _ in range(2)],
            pltpu.SemaphoreType.DMA,
        ],
        compiler_params=pltpu.CompilerParams(use_tc_tiling_on_sc=True,
                                             needs_layout_passes=False),
    )
    outT = run(idx, tokT, posT)
    return outT.transpose(0, 2, 1)


# speedup vs baseline: 2.2971x; 2.2971x over previous
import functools

import jax
import jax.numpy as jnp
from jax import lax
from jax.experimental import pallas as pl
from jax.experimental.pallas import tpu as pltpu
from jax.experimental.pallas import tpu_sc as plsc

LANES = 16
NC, NS = 2, 16
NW = NC * NS
NBB = 8


def _tpe_body(x_hbm, tokT_hbm, posT_hbm, outT_hbm, row_v, posrow_v, xblk,
              oblk, x_sems, o_sems, ld_sem, *, batch, seq, dmodel):
    wid = lax.axis_index("s") * NC + lax.axis_index("c")
    nblk = batch // NBB
    npass = dmodel // NW

    def x_copy(blk, par):
        return pltpu.make_async_copy(x_hbm.at[pl.ds(blk * NBB, NBB)],
                                     xblk[par], x_sems[par])

    for p in range(npass):
        d = wid + p * NW

        pltpu.async_copy(tokT_hbm.at[d], row_v, ld_sem)
        pltpu.make_async_copy(tokT_hbm.at[d], row_v, ld_sem).wait()
        pltpu.async_copy(posT_hbm.at[d], posrow_v, ld_sem)
        pltpu.make_async_copy(posT_hbm.at[d], posrow_v, ld_sem).wait()

        def o_copy(blk, par, d=d):
            return pltpu.make_async_copy(
                oblk[par], outT_hbm.at[pl.ds(blk * NBB, NBB), d],
                o_sems[par])

        x_copy(0, 0).start()

        @pl.loop(0, nblk, step=2)
        def _pair(blk0, o_copy=o_copy):
            for par in range(2):
                blk = blk0 + par

                @pl.when(blk + 1 < nblk)
                def _prefetch(blk=blk, par=par):
                    x_copy(blk + 1, 1 - par).start()

                x_copy(blk, par).wait()

                @pl.when(blk - 2 >= 0)
                def _drain(blk=blk, par=par, o_copy=o_copy):
                    o_copy(blk - 2, par).wait()

                @plsc.parallel_loop(0, seq // LANES, unroll=2)
                def _sc(sc, par=par):
                    pv = posrow_v[pl.ds(sc * LANES, LANES)]
                    for bb in range(NBB):
                        idx = xblk[par][bb, pl.ds(sc * LANES, LANES)]
                        g = plsc.load_gather(row_v, [idx])
                        oblk[par][bb, pl.ds(sc * LANES, LANES)] = g + pv

                o_copy(blk, par).start()

        for par in range(2):
            o_copy(nblk - 2 + par, par).wait()


def kernel(x, token_table, pos_table):
    batch, seq = x.shape
    vocab, dmodel = token_table.shape

    idx = x.astype(jnp.int32)
    tokT = token_table.T
    posT = pos_table.T
    mesh = plsc.VectorSubcoreMesh(core_axis_name="c", subcore_axis_name="s")

    run = pl.kernel(
        functools.partial(_tpe_body, batch=batch, seq=seq, dmodel=dmodel),
        out_type=jax.ShapeDtypeStruct((batch, dmodel, seq), jnp.float32),
        mesh=mesh,
        scratch_types=[
            pltpu.VMEM((vocab,), jnp.float32),
            pltpu.VMEM((seq,), jnp.float32),
            [pltpu.VMEM((NBB, seq), jnp.int32) for _ in range(2)],
            [pltpu.VMEM((NBB, seq), jnp.float32) for _ in range(2)],
            [pltpu.SemaphoreType.DMA for _ in range(2)],
            [pltpu.SemaphoreType.DMA for _ in range(2)],
            pltpu.SemaphoreType.DMA,
        ],
        compiler_params=pltpu.CompilerParams(use_tc_tiling_on_sc=True,
                                             needs_layout_passes=False),
    )
    outT = run(idx, tokT, posT)
    return outT.transpose(0, 2, 1)


# speedup vs baseline: 2.3114x; 1.0062x over previous
import functools

import jax
import jax.numpy as jnp
from jax import lax
from jax.experimental import pallas as pl
from jax.experimental.pallas import tpu as pltpu
from jax.experimental.pallas import tpu_sc as plsc

LANES = 16
NC, NS = 2, 16
NW = NC * NS
NBB = 8


def _tpe_body(x_hbm, tokT_hbm, posT_hbm, outT_hbm, row_v, posrow_v, xblk,
              oblk, x_sems, o_sems, ld_sem, *, batch, seq, dmodel):
    wid = lax.axis_index("s") * NC + lax.axis_index("c")
    nblk = batch // NBB
    npass = dmodel // NW

    def x_copy(blk, par):
        return pltpu.make_async_copy(x_hbm.at[pl.ds(blk * NBB, NBB)],
                                     xblk[par], x_sems[par])

    for p in range(npass):
        d = wid + p * NW

        pltpu.async_copy(tokT_hbm.at[d], row_v, ld_sem)
        pltpu.make_async_copy(tokT_hbm.at[d], row_v, ld_sem).wait()
        pltpu.async_copy(posT_hbm.at[d], posrow_v, ld_sem)
        pltpu.make_async_copy(posT_hbm.at[d], posrow_v, ld_sem).wait()

        def o_copy(blk, par, d=d):
            return pltpu.make_async_copy(
                oblk[par], outT_hbm.at[pl.ds(blk * NBB, NBB), d],
                o_sems[par])

        x_copy(0, 0).start()

        @pl.loop(0, nblk, step=2)
        def _pair(blk0, o_copy=o_copy):
            for par in range(2):
                blk = blk0 + par

                @pl.when(blk + 1 < nblk)
                def _prefetch(blk=blk, par=par):
                    x_copy(blk + 1, 1 - par).start()

                x_copy(blk, par).wait()

                @pl.when(blk - 2 >= 0)
                def _drain(blk=blk, par=par, o_copy=o_copy):
                    o_copy(blk - 2, par).wait()

                @plsc.parallel_loop(0, seq // LANES, unroll=4)
                def _sc(sc, par=par):
                    pv = posrow_v[pl.ds(sc * LANES, LANES)]
                    for bb in range(NBB):
                        idx = xblk[par][bb, pl.ds(sc * LANES, LANES)]
                        g = plsc.load_gather(row_v, [idx])
                        oblk[par][bb, pl.ds(sc * LANES, LANES)] = g + pv

                o_copy(blk, par).start()

        for par in range(2):
            o_copy(nblk - 2 + par, par).wait()


def kernel(x, token_table, pos_table):
    batch, seq = x.shape
    vocab, dmodel = token_table.shape

    idx = x.astype(jnp.int32)
    tokT = token_table.T
    posT = pos_table.T
    mesh = plsc.VectorSubcoreMesh(core_axis_name="c", subcore_axis_name="s")

    run = pl.kernel(
        functools.partial(_tpe_body, batch=batch, seq=seq, dmodel=dmodel),
        out_type=jax.ShapeDtypeStruct((batch, dmodel, seq), jnp.float32),
        mesh=mesh,
        scratch_types=[
            pltpu.VMEM((vocab,), jnp.float32),
            pltpu.VMEM((seq,), jnp.float32),
            [pltpu.VMEM((NBB, seq), jnp.int32) for _ in range(2)],
            [pltpu.VMEM((NBB, seq), jnp.float32) for _ in range(2)],
            [pltpu.SemaphoreType.DMA for _ in range(2)],
            [pltpu.SemaphoreType.DMA for _ in range(2)],
            pltpu.SemaphoreType.DMA,
        ],
        compiler_params=pltpu.CompilerParams(use_tc_tiling_on_sc=True,
                                             needs_layout_passes=False),
    )
    outT = run(idx, tokT, posT)
    return outT.transpose(0, 2, 1)
